# baseline (speedup 1.0000x reference)
import functools

import jax
import jax.numpy as jnp
from jax import lax
from jax.experimental import pallas as pl
from jax.experimental.pallas import tpu as pltpu
from jax.experimental.pallas import tpu_sc as plsc

BATCH = 4096
SEQ = 200
EMBED = 64
TOTAL = BATCH * SEQ

NC = 2
NS = 16
NW = NC * NS
PER_W = TOTAL // NW
C = 128
G = PER_W // C
NBUF = 4

_mesh = plsc.VectorSubcoreMesh(core_axis_name="c", subcore_axis_name="s")


def _body(idx_hbm, table_hbm, out_hbm, idx_v, b0, b1, b2, b3, s0, s1, s2, s3):
    bufs = (b0, b1, b2, b3)
    sems = (s0, s1, s2, s3)
    wid = lax.axis_index("s") * NC + lax.axis_index("c")
    gbase = wid * G
    rbase = wid * PER_W

    pltpu.sync_copy(idx_hbm.at[pl.ds(gbase, G)], idx_v)

    for b in range(NBUF):
        pltpu.async_copy(table_hbm.at[idx_v.at[b]], bufs[b], sems[b])

    def step(t, carry):
        for b in range(NBUF):
            g = t * NBUF + b
            pltpu.make_async_copy(table_hbm.at[idx_v.at[g]], bufs[b], sems[b]).wait()
            pltpu.sync_copy(bufs[b], out_hbm.at[pl.ds(rbase + g * C, C)])
            pltpu.async_copy(table_hbm.at[idx_v.at[g + NBUF]], bufs[b], sems[b])
        return carry

    lax.fori_loop(0, G // NBUF - 1, step, 0)

    for b in range(NBUF):
        g = G - NBUF + b
        pltpu.make_async_copy(table_hbm.at[idx_v.at[g]], bufs[b], sems[b]).wait()
        pltpu.sync_copy(bufs[b], out_hbm.at[pl.ds(rbase + g * C, C)])


@jax.jit
def _gather(idx2d, table):
    run = pl.kernel(
        _body,
        out_type=jax.ShapeDtypeStruct((TOTAL, EMBED), jnp.float32),
        mesh=_mesh,
        scratch_types=[
            pltpu.VMEM((G, C), jnp.int32),
        ] + [pltpu.VMEM((C, EMBED), jnp.float32) for _ in range(NBUF)]
          + [pltpu.SemaphoreType.DMA for _ in range(NBUF)],
        compiler_params=pltpu.CompilerParams(use_tc_tiling_on_sc=False),
    )
    return run(idx2d, table)


def kernel(word_batch, table):
    idx2d = word_batch.astype(jnp.int32).reshape(NW * G, C)
    out = _gather(idx2d, table)
    return out.reshape(BATCH, SEQ, EMBED)
